# baseline (speedup 1.0000x reference)
import jax
import jax.numpy as jnp
from jax import lax
from jax.experimental import pallas as pl
from jax.experimental.pallas import tpu as pltpu
from jax.experimental.pallas import tpu_sc as plsc

_NC, _NS, _NL = 2, 16, 16
_NW = _NC * _NS
_PADW = 640


def _sim_block(tq_ref, dq_ref, out_ref):
    out_ref[...] = lax.dot_general(
        tq_ref[...], dq_ref[...],
        dimension_numbers=(((1,), (1,)), ((), ())),
        preferred_element_type=jnp.float32,
    )


def _sim_matmul(tq, dq, bt, bd):
    t, dim = tq.shape
    d = dq.shape[0]
    return pl.pallas_call(
        _sim_block,
        grid=(t // bt, d // bd),
        in_specs=[
            pl.BlockSpec((bt, dim), lambda i, j: (i, 0)),
            pl.BlockSpec((bd, dim), lambda i, j: (j, 0)),
        ],
        out_specs=pl.BlockSpec((bt, bd), lambda i, j: (i, j)),
        out_shape=jax.ShapeDtypeStruct((t, d), jnp.float32),
    )(tq, dq)


def _convert_position(pos):
    cx = 0.5 * (pos[:, 0] + pos[:, 2])
    cy = 0.5 * (pos[:, 1] + pos[:, 3])
    w = jnp.maximum(pos[:, 2] - pos[:, 0], 1e-6)
    h = jnp.maximum(pos[:, 3] - pos[:, 1], 1e-6)
    return jnp.stack([cx, cy, w, h], axis=1)


def _normalize(x):
    n = jnp.maximum(jnp.linalg.norm(x, axis=1, keepdims=True), 1e-12)
    return x / n


def _make_edge_kernel(t, d, dim, k, e):
    epw = e // _NW
    bsz = 64
    nb = epw // bsz
    spb = bsz // k
    row_w = dim + 4
    nch = row_w // _NL + 1

    def body(tp_hbm, dp_hbm, idx_hbm, out_hbm,
             idx_v, det_v, tf_v, st1, st2, sem):
        wid = lax.axis_index("s") * _NC + lax.axis_index("c")
        wbase = wid * epw
        lanes = lax.iota(jnp.int32, _NL)
        tailmask = lanes < 4
        mneg = jnp.where(tailmask, -1.0, 1.0)

        def batch_body(b, carry):
            gbase = wbase + b * bsz
            pltpu.sync_copy(idx_hbm.at[pl.ds(gbase, bsz)], idx_v)
            pltpu.async_copy(dp_hbm.at[idx_v], det_v, sem).wait()
            src0 = gbase // k
            pltpu.sync_copy(tp_hbm.at[pl.ds(src0 * _PADW, spb * _PADW)], tf_v)
            for s in range(spb):
                tvals = [tf_v[pl.ds(s * _PADW + _NL * m, _NL)]
                         for m in range(nch)]

                def edge_body(ei, c2):
                    row = s * k + ei
                    rbase = row * row_w
                    for m in range(nch):
                        v = det_v[row, pl.ds(_NL * m, _NL)] + tvals[m]
                        st1[pl.ds(rbase + _NL * m, _NL)] = v
                        st2[pl.ds(rbase + _NL * m, _NL)] = (
                            v * mneg if m == 0 else v)
                    return c2

                lax.fori_loop(0, k, edge_body, 0)
            pltpu.sync_copy(st1.at[pl.ds(0, bsz * row_w)],
                            out_hbm.at[pl.ds(gbase * row_w, bsz * row_w)])
            pltpu.sync_copy(st2.at[pl.ds(0, bsz * row_w)],
                            out_hbm.at[pl.ds((e + gbase) * row_w,
                                             bsz * row_w)])
            return carry

        lax.fori_loop(0, nb, batch_body, 0)

    mesh = plsc.VectorSubcoreMesh(core_axis_name="c", subcore_axis_name="s")
    return pl.kernel(
        body,
        out_type=jax.ShapeDtypeStruct((2 * e * row_w,), jnp.float32),
        mesh=mesh,
        scratch_types=[
            pltpu.VMEM((bsz,), jnp.int32),
            pltpu.VMEM((bsz, _PADW), jnp.float32),
            pltpu.VMEM((spb * _PADW,), jnp.float32),
            pltpu.VMEM((bsz * row_w + _NL,), jnp.float32),
            pltpu.VMEM((bsz * row_w + _NL,), jnp.float32),
            pltpu.SemaphoreType.DMA,
        ],
    )


def kernel(tracklet_feat, det_feat, tracklet_pos, det_pos, img_w, img_h):
    t, dim = tracklet_feat.shape
    d = det_feat.shape[0]
    k = min(32, d)
    e = t * k
    tq = _normalize(tracklet_feat)
    dq = _normalize(det_feat)
    sim = _sim_matmul(tq, dq, min(256, t), min(2048, d))
    _, idx = lax.top_k(sim, k)

    tp = _convert_position(tracklet_pos)
    dp = _convert_position(det_pos)
    iw = jnp.asarray(img_w, jnp.float32)
    ih = jnp.asarray(img_h, jnp.float32)
    ta = jnp.stack([tp[:, 0] / iw, tp[:, 1] / ih,
                    jnp.log(tp[:, 2]), jnp.log(tp[:, 3])], axis=1)
    da = jnp.stack([dp[:, 0] / iw, dp[:, 1] / ih,
                    jnp.log(dp[:, 2]), jnp.log(dp[:, 3])], axis=1)
    tpack = jnp.concatenate(
        [ta, 0.5 * tracklet_feat,
         jnp.zeros((t, _PADW - dim - 4), jnp.float32)], axis=1)
    dpack = jnp.concatenate(
        [-da, 0.5 * det_feat,
         jnp.zeros((d, _PADW - dim - 4), jnp.float32)], axis=1)

    edge_fn = _make_edge_kernel(t, d, dim, k, e)
    flat = edge_fn(tpack.reshape(-1), dpack, idx.reshape(-1))
    return flat.reshape(2 * e, dim + 4)


# speedup vs baseline: 2.0968x; 2.0968x over previous
import functools

import jax
import jax.numpy as jnp
from jax import lax
from jax.experimental import pallas as pl
from jax.experimental.pallas import tpu as pltpu
from jax.experimental.pallas import tpu_sc as plsc

_NC, _NS, _NL = 2, 16, 16
_NW = _NC * _NS
_PADW = 640


def _sim_block(tq_ref, dq_ref, out_ref):
    out_ref[...] = lax.dot_general(
        tq_ref[...], dq_ref[...],
        dimension_numbers=(((1,), (1,)), ((), ())),
        preferred_element_type=jnp.float32,
    )


def _sim_matmul(tq, dq, bt, bd):
    t, dim = tq.shape
    d = dq.shape[0]
    return pl.pallas_call(
        _sim_block,
        grid=(t // bt, d // bd),
        in_specs=[
            pl.BlockSpec((bt, dim), lambda i, j: (i, 0)),
            pl.BlockSpec((bd, dim), lambda i, j: (j, 0)),
        ],
        out_specs=pl.BlockSpec((bt, bd), lambda i, j: (i, j)),
        out_shape=jax.ShapeDtypeStruct((t, d), jnp.float32),
    )(tq, dq)


def _simtopk_body(tq_ref, dq_ref, idx_ref, cand_v, cand_i, *, bd, k, nd):
    bt = tq_ref.shape[0]
    j = pl.program_id(1)
    blk = lax.dot_general(
        tq_ref[...], dq_ref[...],
        dimension_numbers=(((1,), (1,)), ((), ())),
        preferred_element_type=jnp.float32,
    )
    colio = lax.broadcasted_iota(jnp.int32, (bt, bd), 1)
    kio = lax.broadcasted_iota(jnp.int32, (bt, k), 1)
    ninf = jnp.float32(-jnp.inf)

    def it(r, state):
        work, vals, idxs = state
        m = jnp.max(work, axis=1, keepdims=True)
        am = jnp.min(jnp.where(work == m, colio, bd), axis=1, keepdims=True)
        sel_k = kio == r
        vals = jnp.where(sel_k, m, vals)
        idxs = jnp.where(sel_k, am + j * bd, idxs)
        work = jnp.where(colio == am, ninf, work)
        return work, vals, idxs

    _, vals, idxs = lax.fori_loop(
        0, k, it,
        (blk, jnp.zeros((bt, k), jnp.float32), jnp.zeros((bt, k), jnp.int32)),
        unroll=4)
    cand_v[j] = vals
    cand_i[j] = idxs

    @pl.when(j == nd - 1)
    def _final():
        nc = nd * k
        w0 = cand_v[...]
        civ = cand_i[...]
        posio = (lax.broadcasted_iota(jnp.int32, (nd, bt, k), 0) * k
                 + lax.broadcasted_iota(jnp.int32, (nd, bt, k), 2))
        bigi = jnp.int32(2**30)

        def it2(r, state):
            w, oidx = state
            m = jnp.max(jnp.max(w, axis=0), axis=1)
            mb = m[None, :, None]
            am = jnp.min(jnp.min(jnp.where(w == mb, posio, nc), axis=0),
                         axis=1)
            sel = posio == am[None, :, None]
            gidx = jnp.min(jnp.min(jnp.where(sel, civ, bigi), axis=0),
                           axis=1)
            oidx = jnp.where(kio == r, gidx[:, None], oidx)
            w = jnp.where(sel, ninf, w)
            return w, oidx

        _, oidx = lax.fori_loop(0, k, it2,
                                (w0, jnp.zeros((bt, k), jnp.int32)),
                                unroll=4)
        idx_ref[...] = oidx


def _sim_topk(tq, dq, k, bt, bd, interpret=False):
    t, dim = tq.shape
    d = dq.shape[0]
    nd = d // bd
    body = functools.partial(_simtopk_body, bd=bd, k=k, nd=nd)
    return pl.pallas_call(
        body,
        grid=(t // bt, nd),
        in_specs=[
            pl.BlockSpec((bt, dim), lambda i, j: (i, 0)),
            pl.BlockSpec((bd, dim), lambda i, j: (j, 0)),
        ],
        out_specs=pl.BlockSpec((bt, k), lambda i, j: (i, 0)),
        out_shape=jax.ShapeDtypeStruct((t, k), jnp.int32),
        scratch_shapes=[
            pltpu.VMEM((nd, bt, k), jnp.float32),
            pltpu.VMEM((nd, bt, k), jnp.int32),
        ],
        interpret=interpret,
    )(tq, dq)


def _convert_position(pos):
    cx = 0.5 * (pos[:, 0] + pos[:, 2])
    cy = 0.5 * (pos[:, 1] + pos[:, 3])
    w = jnp.maximum(pos[:, 2] - pos[:, 0], 1e-6)
    h = jnp.maximum(pos[:, 3] - pos[:, 1], 1e-6)
    return jnp.stack([cx, cy, w, h], axis=1)


def _normalize(x):
    n = jnp.maximum(jnp.linalg.norm(x, axis=1, keepdims=True), 1e-12)
    return x / n


def _make_edge_kernel(t, d, dim, k, e):
    epw = e // _NW
    bsz = 64
    nb = epw // bsz
    spb = bsz // k
    row_w = dim + 4
    nch = row_w // _NL + 1

    def body(tp_hbm, dp_hbm, idx_hbm, out_hbm,
             idx_v, det_v, tf_v, st1, st2, sem):
        wid = lax.axis_index("s") * _NC + lax.axis_index("c")
        wbase = wid * epw
        lanes = lax.iota(jnp.int32, _NL)
        tailmask = lanes < 4
        mneg = jnp.where(tailmask, -1.0, 1.0)

        def batch_body(b, carry):
            gbase = wbase + b * bsz
            pltpu.sync_copy(idx_hbm.at[pl.ds(gbase, bsz)], idx_v)
            pltpu.async_copy(dp_hbm.at[idx_v], det_v, sem).wait()
            src0 = gbase // k
            pltpu.sync_copy(tp_hbm.at[pl.ds(src0 * _PADW, spb * _PADW)], tf_v)
            for s in range(spb):
                tvals = [tf_v[pl.ds(s * _PADW + _NL * m, _NL)]
                         for m in range(nch)]

                def edge_body(ei, c2):
                    row = s * k + ei
                    rbase = row * row_w
                    for m in range(nch):
                        v = det_v[row, pl.ds(_NL * m, _NL)] + tvals[m]
                        st1[pl.ds(rbase + _NL * m, _NL)] = v
                        st2[pl.ds(rbase + _NL * m, _NL)] = (
                            v * mneg if m == 0 else v)
                    return c2

                lax.fori_loop(0, k, edge_body, 0)
            pltpu.sync_copy(st1.at[pl.ds(0, bsz * row_w)],
                            out_hbm.at[pl.ds(gbase * row_w, bsz * row_w)])
            pltpu.sync_copy(st2.at[pl.ds(0, bsz * row_w)],
                            out_hbm.at[pl.ds((e + gbase) * row_w,
                                             bsz * row_w)])
            return carry

        lax.fori_loop(0, nb, batch_body, 0)

    mesh = plsc.VectorSubcoreMesh(core_axis_name="c", subcore_axis_name="s")
    return pl.kernel(
        body,
        out_type=jax.ShapeDtypeStruct((2 * e * row_w,), jnp.float32),
        mesh=mesh,
        scratch_types=[
            pltpu.VMEM((bsz,), jnp.int32),
            pltpu.VMEM((bsz, _PADW), jnp.float32),
            pltpu.VMEM((spb * _PADW,), jnp.float32),
            pltpu.VMEM((bsz * row_w + _NL,), jnp.float32),
            pltpu.VMEM((bsz * row_w + _NL,), jnp.float32),
            pltpu.SemaphoreType.DMA,
        ],
    )


def kernel(tracklet_feat, det_feat, tracklet_pos, det_pos, img_w, img_h):
    t, dim = tracklet_feat.shape
    d = det_feat.shape[0]
    k = min(32, d)
    e = t * k
    tq = _normalize(tracklet_feat)
    dq = _normalize(det_feat)
    idx = _sim_topk(tq, dq, k, min(256, t), min(2048, d))

    tp = _convert_position(tracklet_pos)
    dp = _convert_position(det_pos)
    iw = jnp.asarray(img_w, jnp.float32)
    ih = jnp.asarray(img_h, jnp.float32)
    ta = jnp.stack([tp[:, 0] / iw, tp[:, 1] / ih,
                    jnp.log(tp[:, 2]), jnp.log(tp[:, 3])], axis=1)
    da = jnp.stack([dp[:, 0] / iw, dp[:, 1] / ih,
                    jnp.log(dp[:, 2]), jnp.log(dp[:, 3])], axis=1)
    tpack = jnp.concatenate(
        [ta, 0.5 * tracklet_feat,
         jnp.zeros((t, _PADW - dim - 4), jnp.float32)], axis=1)
    dpack = jnp.concatenate(
        [-da, 0.5 * det_feat,
         jnp.zeros((d, _PADW - dim - 4), jnp.float32)], axis=1)

    edge_fn = _make_edge_kernel(t, d, dim, k, e)
    flat = edge_fn(tpack.reshape(-1), dpack, idx.reshape(-1))
    return flat.reshape(2 * e, dim + 4)


# speedup vs baseline: 2.1837x; 1.0414x over previous
import functools

import jax
import jax.numpy as jnp
from jax import lax
from jax.experimental import pallas as pl
from jax.experimental.pallas import tpu as pltpu
from jax.experimental.pallas import tpu_sc as plsc

_NC, _NS, _NL = 2, 16, 16
_NW = _NC * _NS
_PADW = 640


def _sim_block(tq_ref, dq_ref, out_ref):
    out_ref[...] = lax.dot_general(
        tq_ref[...], dq_ref[...],
        dimension_numbers=(((1,), (1,)), ((), ())),
        preferred_element_type=jnp.float32,
    )


def _sim_matmul(tq, dq, bt, bd):
    t, dim = tq.shape
    d = dq.shape[0]
    return pl.pallas_call(
        _sim_block,
        grid=(t // bt, d // bd),
        in_specs=[
            pl.BlockSpec((bt, dim), lambda i, j: (i, 0)),
            pl.BlockSpec((bd, dim), lambda i, j: (j, 0)),
        ],
        out_specs=pl.BlockSpec((bt, bd), lambda i, j: (i, j)),
        out_shape=jax.ShapeDtypeStruct((t, d), jnp.float32),
    )(tq, dq)


def _simtopk_body(tq_ref, dq_ref, idx_ref, cand_v, cand_i, *, bd, k, nd):
    bt = tq_ref.shape[0]
    j = pl.program_id(1)
    blk = lax.dot_general(
        tq_ref[...], dq_ref[...],
        dimension_numbers=(((1,), (1,)), ((), ())),
        preferred_element_type=jnp.float32,
    )
    colio = lax.broadcasted_iota(jnp.int32, (bt, bd), 1)
    kio = lax.broadcasted_iota(jnp.int32, (bt, k), 1)
    ninf = jnp.float32(-jnp.inf)

    def it(r, state):
        work, vals, idxs = state
        m = jnp.max(work, axis=1, keepdims=True)
        am = jnp.min(jnp.where(work == m, colio, bd), axis=1, keepdims=True)
        sel_k = kio == r
        vals = jnp.where(sel_k, m, vals)
        idxs = jnp.where(sel_k, am + j * bd, idxs)
        work = jnp.where(colio == am, ninf, work)
        return work, vals, idxs

    _, vals, idxs = lax.fori_loop(
        0, k, it,
        (blk, jnp.zeros((bt, k), jnp.float32), jnp.zeros((bt, k), jnp.int32)),
        unroll=4)
    cand_v[j] = vals
    cand_i[j] = idxs

    @pl.when(j == nd - 1)
    def _final():
        nc = nd * k
        w0 = cand_v[...]
        civ = cand_i[...]
        posio = (lax.broadcasted_iota(jnp.int32, (nd, bt, k), 0) * k
                 + lax.broadcasted_iota(jnp.int32, (nd, bt, k), 2))
        bigi = jnp.int32(2**30)

        def it2(r, state):
            w, oidx = state
            m = jnp.max(jnp.max(w, axis=0), axis=1)
            mb = m[None, :, None]
            am = jnp.min(jnp.min(jnp.where(w == mb, posio, nc), axis=0),
                         axis=1)
            sel = posio == am[None, :, None]
            gidx = jnp.min(jnp.min(jnp.where(sel, civ, bigi), axis=0),
                           axis=1)
            oidx = jnp.where(kio == r, gidx[:, None], oidx)
            w = jnp.where(sel, ninf, w)
            return w, oidx

        _, oidx = lax.fori_loop(0, k, it2,
                                (w0, jnp.zeros((bt, k), jnp.int32)),
                                unroll=4)
        idx_ref[...] = oidx


def _sim_topk(tq, dq, k, bt, bd, interpret=False):
    t, dim = tq.shape
    d = dq.shape[0]
    nd = d // bd
    body = functools.partial(_simtopk_body, bd=bd, k=k, nd=nd)
    return pl.pallas_call(
        body,
        grid=(t // bt, nd),
        in_specs=[
            pl.BlockSpec((bt, dim), lambda i, j: (i, 0)),
            pl.BlockSpec((bd, dim), lambda i, j: (j, 0)),
        ],
        out_specs=pl.BlockSpec((bt, k), lambda i, j: (i, 0)),
        out_shape=jax.ShapeDtypeStruct((t, k), jnp.int32),
        scratch_shapes=[
            pltpu.VMEM((nd, bt, k), jnp.float32),
            pltpu.VMEM((nd, bt, k), jnp.int32),
        ],
        interpret=interpret,
    )(tq, dq)


def _convert_position(pos):
    cx = 0.5 * (pos[:, 0] + pos[:, 2])
    cy = 0.5 * (pos[:, 1] + pos[:, 3])
    w = jnp.maximum(pos[:, 2] - pos[:, 0], 1e-6)
    h = jnp.maximum(pos[:, 3] - pos[:, 1], 1e-6)
    return jnp.stack([cx, cy, w, h], axis=1)


def _normalize(x):
    n = jnp.maximum(jnp.linalg.norm(x, axis=1, keepdims=True), 1e-12)
    return x / n


def _make_edge_kernel(t, d, dim, k, e):
    epw = e // _NW
    bsz = 32
    nb = epw // bsz
    row_w = dim + 4
    nch = row_w // _NL + 1

    def body(tp_hbm, dp_hbm, idx_hbm, out_hbm,
             idx0, idx1, det0, det1, tf0, tf1,
             st1a, st1b, st2a, st2b, gs0, gs1, os0, os1):
        wid = lax.axis_index("s") * _NC + lax.axis_index("c")
        wbase = wid * epw
        lanes = lax.iota(jnp.int32, _NL)
        mneg = jnp.where(lanes < 4, -1.0, 1.0)
        idxs, dets, tfs = (idx0, idx1), (det0, det1), (tf0, tf1)
        s1s, s2s = (st1a, st1b), (st2a, st2b)
        gss, oss = (gs0, gs1), (os0, os1)
        nw = bsz * row_w

        def prefetch(b, ph):
            gb = wbase + b * bsz
            pltpu.sync_copy(idx_hbm.at[pl.ds(gb, bsz)], idxs[ph])
            pltpu.async_copy(dp_hbm.at[idxs[ph]], dets[ph], gss[ph])

        prefetch(0, 0)

        def pair_body(g2, carry):
            for ph in range(2):
                b = g2 * 2 + ph
                gbase = wbase + b * bsz
                src0 = gbase // k

                @pl.when(b < nb - 1)
                def _():
                    prefetch(b + 1, 1 - ph)

                pltpu.sync_copy(tp_hbm.at[pl.ds(src0 * _PADW, _PADW)],
                                tfs[ph])
                pltpu.make_async_copy(dp_hbm.at[idxs[ph]], dets[ph],
                                      gss[ph]).wait()

                @pl.when(b >= 2)
                def _():
                    pltpu.make_async_copy(
                        s1s[ph].at[pl.ds(0, nw)],
                        out_hbm.at[pl.ds(0, nw)], oss[ph]).wait()
                    pltpu.make_async_copy(
                        s1s[ph].at[pl.ds(0, nw)],
                        out_hbm.at[pl.ds(0, nw)], oss[ph]).wait()

                tvals = [tfs[ph][pl.ds(_NL * m, _NL)] for m in range(nch)]

                def edge_body(ei, c2):
                    rbase = ei * row_w
                    for m in range(nch):
                        v = dets[ph][ei, pl.ds(_NL * m, _NL)] + tvals[m]
                        s1s[ph][pl.ds(rbase + _NL * m, _NL)] = v
                        s2s[ph][pl.ds(rbase + _NL * m, _NL)] = (
                            v * mneg if m == 0 else v)
                    return c2

                lax.fori_loop(0, bsz, edge_body, 0)
                pltpu.async_copy(s1s[ph].at[pl.ds(0, nw)],
                                 out_hbm.at[pl.ds(gbase * row_w, nw)],
                                 oss[ph])
                pltpu.async_copy(s2s[ph].at[pl.ds(0, nw)],
                                 out_hbm.at[pl.ds((e + gbase) * row_w, nw)],
                                 oss[ph])
            return carry

        lax.fori_loop(0, nb // 2, pair_body, 0)
        for ph in range(2):
            for _u in range(2):
                pltpu.make_async_copy(s1s[ph].at[pl.ds(0, nw)],
                                      out_hbm.at[pl.ds(0, nw)],
                                      oss[ph]).wait()

    mesh = plsc.VectorSubcoreMesh(core_axis_name="c", subcore_axis_name="s")
    return pl.kernel(
        body,
        out_type=jax.ShapeDtypeStruct((2 * e * row_w,), jnp.float32),
        mesh=mesh,
        scratch_types=[
            pltpu.VMEM((bsz,), jnp.int32),
            pltpu.VMEM((bsz,), jnp.int32),
            pltpu.VMEM((bsz, _PADW), jnp.float32),
            pltpu.VMEM((bsz, _PADW), jnp.float32),
            pltpu.VMEM((_PADW,), jnp.float32),
            pltpu.VMEM((_PADW,), jnp.float32),
            pltpu.VMEM((bsz * row_w + _NL,), jnp.float32),
            pltpu.VMEM((bsz * row_w + _NL,), jnp.float32),
            pltpu.VMEM((bsz * row_w + _NL,), jnp.float32),
            pltpu.VMEM((bsz * row_w + _NL,), jnp.float32),
            pltpu.SemaphoreType.DMA,
            pltpu.SemaphoreType.DMA,
            pltpu.SemaphoreType.DMA,
            pltpu.SemaphoreType.DMA,
        ],
    )


def kernel(tracklet_feat, det_feat, tracklet_pos, det_pos, img_w, img_h):
    t, dim = tracklet_feat.shape
    d = det_feat.shape[0]
    k = min(32, d)
    e = t * k
    tq = _normalize(tracklet_feat)
    dq = _normalize(det_feat)
    idx = _sim_topk(tq, dq, k, min(256, t), min(2048, d))

    tp = _convert_position(tracklet_pos)
    dp = _convert_position(det_pos)
    iw = jnp.asarray(img_w, jnp.float32)
    ih = jnp.asarray(img_h, jnp.float32)
    ta = jnp.stack([tp[:, 0] / iw, tp[:, 1] / ih,
                    jnp.log(tp[:, 2]), jnp.log(tp[:, 3])], axis=1)
    da = jnp.stack([dp[:, 0] / iw, dp[:, 1] / ih,
                    jnp.log(dp[:, 2]), jnp.log(dp[:, 3])], axis=1)
    tpack = jnp.concatenate(
        [ta, 0.5 * tracklet_feat,
         jnp.zeros((t, _PADW - dim - 4), jnp.float32)], axis=1)
    dpack = jnp.concatenate(
        [-da, 0.5 * det_feat,
         jnp.zeros((d, _PADW - dim - 4), jnp.float32)], axis=1)

    edge_fn = _make_edge_kernel(t, d, dim, k, e)
    flat = edge_fn(tpack.reshape(-1), dpack, idx.reshape(-1))
    return flat.reshape(2 * e, dim + 4)
